# baseline (speedup 1.0000x reference)
import functools

import jax
import jax.numpy as jnp
from jax import lax
from jax.experimental import pallas as pl
from jax.experimental.pallas import tpu as pltpu
from jax.experimental.pallas import tpu_sc as plsc

_L = 200
_D = 64
_DP = 128
_CHUNK = 40
_NPH = _L // _CHUNK
_NBUF = 4
_NC = 2
_NS = 16
_NW = _NC * _NS


def _gather_body(idx_hbm, tok_hbm, pos_hbm, out_hbm,
                 idx_v, obuf_v, pos_sh, psem, gsem, osem):
    n = idx_hbm.shape[0]
    n_per_w = n // _NW
    n_chunks = n_per_w // _CHUNK

    cid = lax.axis_index("c")
    sid = lax.axis_index("s")
    wid = sid * _NC + cid
    base = wid * n_per_w

    pltpu.sync_copy(idx_hbm.at[pl.ds(base, n_per_w)], idx_v)

    @pl.when(sid == 0)
    def _():
        for ph in range(_NPH):
            sl = pl.ds(ph * _CHUNK, _CHUNK)
            pltpu.sync_copy(pos_hbm.at[sl, :], obuf_v.at[0])
            pltpu.sync_copy(obuf_v.at[0], pos_sh.at[sl, :])

    plsc.subcore_barrier()

    def prefill_start(b, c):
        p0 = lax.rem(c, _NPH) * _CHUNK
        pltpu.make_async_copy(
            pos_sh.at[pl.ds(p0, _CHUNK), :], obuf_v.at[b], psem.at[b]).start()

    def prefill_wait(b):
        pltpu.make_async_copy(
            pos_sh.at[pl.ds(0, _CHUNK), :], obuf_v.at[b], psem.at[b]).wait()

    def gadd_start(b, c):
        pltpu.async_copy(
            tok_hbm.at[idx_v.at[pl.ds(c * _CHUNK, _CHUNK)]],
            obuf_v.at[b], gsem.at[b], add=True)

    def gadd_wait(b):
        pltpu.make_async_copy(
            tok_hbm.at[idx_v.at[pl.ds(0, _CHUNK)]],
            obuf_v.at[b], gsem.at[b]).wait()

    def out_start(b, c):
        pltpu.make_async_copy(
            obuf_v.at[b],
            out_hbm.at[pl.ds(base + c * _CHUNK, _CHUNK), :],
            osem.at[b]).start()

    def out_wait(b):
        pltpu.make_async_copy(
            obuf_v.at[b],
            out_hbm.at[pl.ds(0, _CHUNK), :], osem.at[b]).wait()

    n_steps = n_chunks + 2
    n_groups = (n_steps + _NBUF - 1) // _NBUF

    def group_body(g, carry):
        for b in range(_NBUF):
            i = g * _NBUF + b
            bg = (b + _NBUF - 1) % _NBUF
            bo = (b + _NBUF - 2) % _NBUF

            @pl.when(jnp.logical_and(i >= _NBUF, i < n_chunks))
            def _():
                out_wait(b)

            @pl.when(i < n_chunks)
            def _():
                prefill_start(b, i)

            @pl.when(jnp.logical_and(i >= 1, i <= n_chunks))
            def _():
                prefill_wait(bg)
                gadd_start(bg, i - 1)

            @pl.when(jnp.logical_and(i >= 2, i <= n_chunks + 1))
            def _():
                gadd_wait(bo)
                out_start(bo, i - 2)
        return carry

    lax.fori_loop(0, n_groups, group_body, 0)

    for b in range(_NBUF):
        out_wait(b)


_TBLK = 512


def _transpose_pad_body(src_ref, dst_ref):
    x = src_ref[...]
    dst_ref[:, 0:_D] = jnp.transpose(x, (1, 0))
    dst_ref[:, _D:_DP] = jnp.zeros((_TBLK, _DP - _D), jnp.float32)


def _transpose_pad(tok_t):
    v = tok_t.shape[1]
    grid = pl.cdiv(v, _TBLK)
    return pl.pallas_call(
        _transpose_pad_body,
        grid=(grid,),
        in_specs=[pl.BlockSpec((_D, _TBLK), lambda i: (0, i))],
        out_specs=pl.BlockSpec((_TBLK, _DP), lambda i: (i, 0)),
        out_shape=jax.ShapeDtypeStruct((v, _DP), jnp.float32),
    )(tok_t)


def kernel(inputs, token_table, position_table):
    b, l = inputs.shape
    d = token_table.shape[-1]
    n = b * l
    idx_flat = inputs.reshape(n).astype(jnp.int32)
    tok_pad = _transpose_pad(token_table.T)
    pos_pad = jnp.pad(position_table, ((0, 0), (0, _DP - d)))

    grid_kernel = functools.partial(
        pl.kernel,
        mesh=plsc.VectorSubcoreMesh(core_axis_name="c", subcore_axis_name="s"),
        compiler_params=pltpu.CompilerParams(use_tc_tiling_on_sc=True),
        out_type=jax.ShapeDtypeStruct((n, _DP), jnp.float32),
        scratch_types=[
            pltpu.VMEM((n // _NW,), jnp.int32),
            pltpu.VMEM((_NBUF, _CHUNK, _DP), jnp.float32),
            pltpu.VMEM_SHARED((_L, _DP), jnp.float32),
            pltpu.SemaphoreType.DMA((_NBUF,)),
            pltpu.SemaphoreType.DMA((_NBUF,)),
            pltpu.SemaphoreType.DMA((_NBUF,)),
        ],
    )(_gather_body)

    out = grid_kernel(idx_flat, tok_pad, pos_pad)
    return out[:, :d].reshape(b, l, d)


# speedup vs baseline: 2.7071x; 2.7071x over previous
import functools

import jax
import jax.numpy as jnp
from jax import lax
from jax.experimental import pallas as pl
from jax.experimental.pallas import tpu as pltpu
from jax.experimental.pallas import tpu_sc as plsc

_L = 200
_D = 64
_DP = 128
_CHUNK = 40
_NPH = _L // _CHUNK
_NBUF = 4
_NC = 2
_NS = 16
_NW = _NC * _NS


def _gather_body(idx_hbm, tok_hbm, pos_hbm, out_hbm,
                 idx_v, obuf_v, pos_sh, psem, gsem, osem):
    n = idx_hbm.shape[0]
    n_per_w = n // _NW
    n_chunks = n_per_w // _CHUNK

    cid = lax.axis_index("c")
    sid = lax.axis_index("s")
    wid = sid * _NC + cid
    base = wid * n_per_w

    pltpu.sync_copy(idx_hbm.at[pl.ds(base, n_per_w)], idx_v)

    @pl.when(sid == 0)
    def _():
        for ph in range(_NPH):
            sl = pl.ds(ph * _CHUNK, _CHUNK)
            pltpu.sync_copy(pos_hbm.at[sl, :], obuf_v.at[0])
            pltpu.sync_copy(obuf_v.at[0], pos_sh.at[sl, :])

    plsc.subcore_barrier()

    def prefill_start(b, c):
        p0 = lax.rem(c, _NPH) * _CHUNK
        pltpu.make_async_copy(
            pos_sh.at[pl.ds(p0, _CHUNK), :], obuf_v.at[b], psem.at[b]).start()

    def prefill_wait(b):
        pltpu.make_async_copy(
            pos_sh.at[pl.ds(0, _CHUNK), :], obuf_v.at[b], psem.at[b]).wait()

    def gadd_start(b, c):
        pltpu.async_copy(
            tok_hbm.at[idx_v.at[pl.ds(c * _CHUNK, _CHUNK)]],
            obuf_v.at[b], gsem.at[b], add=True)

    def gadd_wait(b):
        pltpu.make_async_copy(
            tok_hbm.at[idx_v.at[pl.ds(0, _CHUNK)]],
            obuf_v.at[b], gsem.at[b]).wait()

    def out_start(b, c):
        pltpu.make_async_copy(
            obuf_v.at[b],
            out_hbm.at[pl.ds(base + c * _CHUNK, _CHUNK), :],
            osem.at[b]).start()

    def out_wait(b):
        pltpu.make_async_copy(
            obuf_v.at[b],
            out_hbm.at[pl.ds(0, _CHUNK), :], osem.at[b]).wait()

    n_steps = n_chunks + 2
    n_groups = (n_steps + _NBUF - 1) // _NBUF

    def group_body(g, carry):
        for b in range(_NBUF):
            i = g * _NBUF + b
            bg = (b + _NBUF - 1) % _NBUF
            bo = (b + _NBUF - 2) % _NBUF

            @pl.when(jnp.logical_and(i >= _NBUF, i < n_chunks))
            def _():
                out_wait(b)

            @pl.when(i < n_chunks)
            def _():
                prefill_start(b, i)

            @pl.when(jnp.logical_and(i >= 1, i <= n_chunks))
            def _():
                prefill_wait(bg)
                gadd_start(bg, i - 1)

            @pl.when(jnp.logical_and(i >= 2, i <= n_chunks + 1))
            def _():
                gadd_wait(bo)
                out_start(bo, i - 2)
        return carry

    lax.fori_loop(0, n_groups, group_body, 0)

    for b in range(_NBUF):
        out_wait(b)


_TBLK = 4096


def _transpose_pad_body(src_ref, dst_ref):
    x = src_ref[...]
    dst_ref[:, 0:_D] = jnp.transpose(x, (1, 0))


def _transpose_pad(tok_t):
    v = tok_t.shape[1]
    grid = pl.cdiv(v, _TBLK)
    return pl.pallas_call(
        _transpose_pad_body,
        grid=(grid,),
        in_specs=[pl.BlockSpec((_D, _TBLK), lambda i: (0, i))],
        out_specs=pl.BlockSpec((_TBLK, _DP), lambda i: (i, 0)),
        out_shape=jax.ShapeDtypeStruct((v, _DP), jnp.float32),
    )(tok_t)


def kernel(inputs, token_table, position_table):
    b, l = inputs.shape
    d = token_table.shape[-1]
    n = b * l
    idx_flat = inputs.reshape(n).astype(jnp.int32)
    tok_pad = _transpose_pad(token_table.T)
    pos_pad = jnp.pad(position_table, ((0, 0), (0, _DP - d)))

    grid_kernel = functools.partial(
        pl.kernel,
        mesh=plsc.VectorSubcoreMesh(core_axis_name="c", subcore_axis_name="s"),
        compiler_params=pltpu.CompilerParams(use_tc_tiling_on_sc=True),
        out_type=jax.ShapeDtypeStruct((n, _DP), jnp.float32),
        scratch_types=[
            pltpu.VMEM((n // _NW,), jnp.int32),
            pltpu.VMEM((_NBUF, _CHUNK, _DP), jnp.float32),
            pltpu.VMEM_SHARED((_L, _DP), jnp.float32),
            pltpu.SemaphoreType.DMA((_NBUF,)),
            pltpu.SemaphoreType.DMA((_NBUF,)),
            pltpu.SemaphoreType.DMA((_NBUF,)),
        ],
    )(_gather_body)

    out = grid_kernel(idx_flat, tok_pad, pos_pad)
    return out[:, :d].reshape(b, l, d)


# speedup vs baseline: 3.1573x; 1.1663x over previous
import functools

import jax
import jax.numpy as jnp
from jax import lax
from jax.experimental import pallas as pl
from jax.experimental.pallas import tpu as pltpu
from jax.experimental.pallas import tpu_sc as plsc

_L = 200
_D = 64
_DP = 128
_CHUNK = 40
_NPH = _L // _CHUNK
_NBUF = 4
_NC = 2
_NS = 16
_NW = _NC * _NS


def _gather_body(idx_hbm, tok_hbm, pos_hbm, out_hbm,
                 idx_v, obuf_v, pos_sh, psem, gsem, osem):
    n = idx_hbm.shape[0]
    n_per_w = n // _NW
    n_chunks = n_per_w // _CHUNK

    cid = lax.axis_index("c")
    sid = lax.axis_index("s")
    wid = sid * _NC + cid
    base = wid * n_per_w

    pltpu.sync_copy(idx_hbm.at[pl.ds(base, n_per_w)], idx_v)

    @pl.when(sid == 0)
    def _():
        for ph in range(_NPH):
            sl = pl.ds(ph * _CHUNK, _CHUNK)
            pltpu.sync_copy(pos_hbm.at[sl, :], obuf_v.at[0])
            pltpu.sync_copy(obuf_v.at[0], pos_sh.at[sl, :])

    plsc.subcore_barrier()

    def prefill_start(b, c):
        p0 = lax.rem(c, _NPH) * _CHUNK
        pltpu.make_async_copy(
            pos_sh.at[pl.ds(p0, _CHUNK), :], obuf_v.at[b], psem.at[b]).start()

    def prefill_wait(b):
        pltpu.make_async_copy(
            pos_sh.at[pl.ds(0, _CHUNK), :], obuf_v.at[b], psem.at[b]).wait()

    def gadd_start(b, c):
        pltpu.async_copy(
            tok_hbm.at[idx_v.at[pl.ds(c * _CHUNK, _CHUNK)]],
            obuf_v.at[b], gsem.at[b], add=True)

    def gadd_wait(b):
        pltpu.make_async_copy(
            tok_hbm.at[idx_v.at[pl.ds(0, _CHUNK)]],
            obuf_v.at[b], gsem.at[b]).wait()

    def out_start(b, c):
        pltpu.make_async_copy(
            obuf_v.at[b],
            out_hbm.at[pl.ds(base + c * _CHUNK, _CHUNK), :],
            osem.at[b]).start()

    def out_wait(b):
        pltpu.make_async_copy(
            obuf_v.at[b],
            out_hbm.at[pl.ds(0, _CHUNK), :], osem.at[b]).wait()

    n_steps = n_chunks + 2
    n_groups = (n_steps + _NBUF - 1) // _NBUF

    def group_body(g, carry):
        for b in range(_NBUF):
            i = g * _NBUF + b
            bg = (b + _NBUF - 1) % _NBUF
            bo = (b + _NBUF - 2) % _NBUF

            @pl.when(jnp.logical_and(i >= _NBUF, i < n_chunks))
            def _():
                out_wait(b)

            @pl.when(i < n_chunks)
            def _():
                prefill_start(b, i)

            @pl.when(jnp.logical_and(i >= 1, i <= n_chunks))
            def _():
                prefill_wait(bg)
                gadd_start(bg, i - 1)

            @pl.when(jnp.logical_and(i >= 2, i <= n_chunks + 1))
            def _():
                gadd_wait(bo)
                out_start(bo, i - 2)
        return carry

    lax.fori_loop(0, n_groups, group_body, 0)

    for b in range(_NBUF):
        out_wait(b)


_TBLK = 8192


def _transpose_pad_body(src_ref, dst_ref):
    x = src_ref[...]
    dst_ref[:, 0:_D] = jnp.transpose(x, (1, 0))


def _transpose_pad(tok_t):
    v = tok_t.shape[1]
    grid = pl.cdiv(v, _TBLK)
    return pl.pallas_call(
        _transpose_pad_body,
        grid=(grid,),
        in_specs=[pl.BlockSpec((_D, _TBLK), lambda i: (0, i))],
        out_specs=pl.BlockSpec((_TBLK, _DP), lambda i: (i, 0)),
        out_shape=jax.ShapeDtypeStruct((v, _DP), jnp.float32),
    )(tok_t)


def kernel(inputs, token_table, position_table):
    b, l = inputs.shape
    d = token_table.shape[-1]
    n = b * l
    idx_flat = inputs.reshape(n).astype(jnp.int32)
    tok_pad = _transpose_pad(token_table.T)
    pos_pad = jnp.pad(position_table, ((0, 0), (0, _DP - d)))

    grid_kernel = functools.partial(
        pl.kernel,
        mesh=plsc.VectorSubcoreMesh(core_axis_name="c", subcore_axis_name="s"),
        compiler_params=pltpu.CompilerParams(use_tc_tiling_on_sc=True),
        out_type=jax.ShapeDtypeStruct((n, _DP), jnp.float32),
        scratch_types=[
            pltpu.VMEM((n // _NW,), jnp.int32),
            pltpu.VMEM((_NBUF, _CHUNK, _DP), jnp.float32),
            pltpu.VMEM_SHARED((_L, _DP), jnp.float32),
            pltpu.SemaphoreType.DMA((_NBUF,)),
            pltpu.SemaphoreType.DMA((_NBUF,)),
            pltpu.SemaphoreType.DMA((_NBUF,)),
        ],
    )(_gather_body)

    out = grid_kernel(idx_flat, tok_pad, pos_pad)
    return out[:, :d].reshape(b, l, d)


# speedup vs baseline: 3.3001x; 1.0452x over previous
import functools

import jax
import jax.numpy as jnp
from jax import lax
from jax.experimental import pallas as pl
from jax.experimental.pallas import tpu as pltpu
from jax.experimental.pallas import tpu_sc as plsc

_L = 200
_D = 64
_DP = 128
_CHUNK = 40
_NPH = _L // _CHUNK
_NBUF = 4
_NC = 2
_NS = 16
_NW = _NC * _NS


def _gather_body(idx_hbm, tok_hbm, pos_hbm, out_hbm,
                 idx_v, obuf_v, pos_sh, psem, gsem, osem):
    n = idx_hbm.shape[0]
    n_per_w = n // _NW
    n_chunks = n_per_w // _CHUNK

    cid = lax.axis_index("c")
    sid = lax.axis_index("s")
    wid = sid * _NC + cid
    base = wid * n_per_w

    pltpu.sync_copy(idx_hbm.at[pl.ds(base, n_per_w)], idx_v)

    @pl.when(sid == 0)
    def _():
        for ph in range(_NPH):
            sl = pl.ds(ph * _CHUNK, _CHUNK)
            pltpu.sync_copy(pos_hbm.at[sl, :], obuf_v.at[0])
            pltpu.sync_copy(obuf_v.at[0], pos_sh.at[sl, :])

    plsc.subcore_barrier()

    def prefill_start(b, c):
        p0 = lax.rem(c, _NPH) * _CHUNK
        pltpu.make_async_copy(
            pos_sh.at[pl.ds(p0, _CHUNK), :], obuf_v.at[b], psem.at[b]).start()

    def prefill_wait(b):
        pltpu.make_async_copy(
            pos_sh.at[pl.ds(0, _CHUNK), :], obuf_v.at[b], psem.at[b]).wait()

    def gadd_start(b, c):
        pltpu.async_copy(
            tok_hbm.at[idx_v.at[pl.ds(c * _CHUNK, _CHUNK)]],
            obuf_v.at[b], gsem.at[b], add=True)

    def gadd_wait(b):
        pltpu.make_async_copy(
            tok_hbm.at[idx_v.at[pl.ds(0, _CHUNK)]],
            obuf_v.at[b], gsem.at[b]).wait()

    def out_start(b, c):
        pltpu.make_async_copy(
            obuf_v.at[b],
            out_hbm.at[pl.ds(base + c * _CHUNK, _CHUNK), :],
            osem.at[b]).start()

    def out_wait(b):
        pltpu.make_async_copy(
            obuf_v.at[b],
            out_hbm.at[pl.ds(0, _CHUNK), :], osem.at[b]).wait()

    n_steps = n_chunks + 2
    n_groups = (n_steps + _NBUF - 1) // _NBUF

    def group_body(g, carry):
        for b in range(_NBUF):
            i = g * _NBUF + b
            bg = (b + _NBUF - 1) % _NBUF
            bo = (b + _NBUF - 2) % _NBUF

            @pl.when(jnp.logical_and(i >= _NBUF, i < n_chunks))
            def _():
                out_wait(b)

            @pl.when(i < n_chunks)
            def _():
                prefill_start(b, i)

            @pl.when(jnp.logical_and(i >= 1, i <= n_chunks))
            def _():
                prefill_wait(bg)
                gadd_start(bg, i - 1)

            @pl.when(jnp.logical_and(i >= 2, i <= n_chunks + 1))
            def _():
                gadd_wait(bo)
                out_start(bo, i - 2)
        return carry

    lax.fori_loop(0, n_groups, group_body, 0)

    for b in range(_NBUF):
        out_wait(b)


_TBLK = 16384


def _transpose_pad_body(src_ref, dst_ref):
    x = src_ref[...]
    dst_ref[:, 0:_D] = jnp.transpose(x, (1, 0))


def _transpose_pad(tok_t):
    v = tok_t.shape[1]
    grid = pl.cdiv(v, _TBLK)
    return pl.pallas_call(
        _transpose_pad_body,
        grid=(grid,),
        in_specs=[pl.BlockSpec((_D, _TBLK), lambda i: (0, i))],
        out_specs=pl.BlockSpec((_TBLK, _DP), lambda i: (i, 0)),
        out_shape=jax.ShapeDtypeStruct((v, _DP), jnp.float32),
    )(tok_t)


def kernel(inputs, token_table, position_table):
    b, l = inputs.shape
    d = token_table.shape[-1]
    n = b * l
    idx_flat = inputs.reshape(n).astype(jnp.int32)
    tok_pad = _transpose_pad(token_table.T)
    pos_pad = jnp.pad(position_table, ((0, 0), (0, _DP - d)))

    grid_kernel = functools.partial(
        pl.kernel,
        mesh=plsc.VectorSubcoreMesh(core_axis_name="c", subcore_axis_name="s"),
        compiler_params=pltpu.CompilerParams(use_tc_tiling_on_sc=True),
        out_type=jax.ShapeDtypeStruct((n, _DP), jnp.float32),
        scratch_types=[
            pltpu.VMEM((n // _NW,), jnp.int32),
            pltpu.VMEM((_NBUF, _CHUNK, _DP), jnp.float32),
            pltpu.VMEM_SHARED((_L, _DP), jnp.float32),
            pltpu.SemaphoreType.DMA((_NBUF,)),
            pltpu.SemaphoreType.DMA((_NBUF,)),
            pltpu.SemaphoreType.DMA((_NBUF,)),
        ],
    )(_gather_body)

    out = grid_kernel(idx_flat, tok_pad, pos_pad)
    return out[:, :d].reshape(b, l, d)


# speedup vs baseline: 3.4921x; 1.0582x over previous
import functools

import jax
import jax.numpy as jnp
from jax import lax
from jax.experimental import pallas as pl
from jax.experimental.pallas import tpu as pltpu
from jax.experimental.pallas import tpu_sc as plsc

_L = 200
_D = 64
_DP = 128
_CHUNK = 128
_LREP = 3200
_NBUF = 4
_NC = 2
_NS = 16
_NW = _NC * _NS


def _gather_body(idx_hbm, tok_hbm, pos_hbm, out_hbm,
                 idx_v, obuf_v, pos_sh, psem, gsem, osem):
    n = idx_hbm.shape[0]
    n_per_w = n // _NW
    n_chunks = n_per_w // _CHUNK

    cid = lax.axis_index("c")
    sid = lax.axis_index("s")
    wid = sid * _NC + cid
    base = wid * n_per_w

    pltpu.sync_copy(idx_hbm.at[pl.ds(base, n_per_w)], idx_v)

    for off, ln in ((0, 104), (104, 96)):
        tmp = obuf_v.at[0, pl.ds(0, ln), :]
        pltpu.sync_copy(pos_hbm.at[pl.ds(off, ln), :], tmp)
        pltpu.sync_copy(tmp, pos_sh.at[pl.ds(sid * _L + off, ln), :])

    plsc.subcore_barrier()

    def prefill_start(b, c):
        p0 = lax.rem(c * _CHUNK, _LREP)
        pltpu.make_async_copy(
            pos_sh.at[pl.ds(p0, _CHUNK), :], obuf_v.at[b], psem.at[b]).start()

    def prefill_wait(b):
        pltpu.make_async_copy(
            pos_sh.at[pl.ds(0, _CHUNK), :], obuf_v.at[b], psem.at[b]).wait()

    def gadd_start(b, c):
        pltpu.async_copy(
            tok_hbm.at[idx_v.at[pl.ds(c * _CHUNK, _CHUNK)]],
            obuf_v.at[b], gsem.at[b], add=True)

    def gadd_wait(b):
        pltpu.make_async_copy(
            tok_hbm.at[idx_v.at[pl.ds(0, _CHUNK)]],
            obuf_v.at[b], gsem.at[b]).wait()

    def out_start(b, c):
        pltpu.make_async_copy(
            obuf_v.at[b],
            out_hbm.at[pl.ds(base + c * _CHUNK, _CHUNK), :],
            osem.at[b]).start()

    def out_wait(b):
        pltpu.make_async_copy(
            obuf_v.at[b],
            out_hbm.at[pl.ds(0, _CHUNK), :], osem.at[b]).wait()

    n_steps = n_chunks + 2
    n_groups = (n_steps + _NBUF - 1) // _NBUF

    def group_body(g, carry):
        for b in range(_NBUF):
            i = g * _NBUF + b
            bg = (b + _NBUF - 1) % _NBUF
            bo = (b + _NBUF - 2) % _NBUF

            @pl.when(jnp.logical_and(i >= _NBUF, i < n_chunks))
            def _():
                out_wait(b)

            @pl.when(i < n_chunks)
            def _():
                prefill_start(b, i)

            @pl.when(jnp.logical_and(i >= 1, i <= n_chunks))
            def _():
                prefill_wait(bg)
                gadd_start(bg, i - 1)

            @pl.when(jnp.logical_and(i >= 2, i <= n_chunks + 1))
            def _():
                gadd_wait(bo)
                out_start(bo, i - 2)
        return carry

    lax.fori_loop(0, n_groups, group_body, 0)

    for b in range(_NBUF):
        out_wait(b)


_TBLK = 16384


def _transpose_pad_body(src_ref, dst_ref):
    x = src_ref[...]
    dst_ref[:, 0:_D] = jnp.transpose(x, (1, 0))


def _transpose_pad(tok_t):
    v = tok_t.shape[1]
    grid = pl.cdiv(v, _TBLK)
    return pl.pallas_call(
        _transpose_pad_body,
        grid=(grid,),
        in_specs=[pl.BlockSpec((_D, _TBLK), lambda i: (0, i))],
        out_specs=pl.BlockSpec((_TBLK, _DP), lambda i: (i, 0)),
        out_shape=jax.ShapeDtypeStruct((v, _DP), jnp.float32),
    )(tok_t)


def kernel(inputs, token_table, position_table):
    b, l = inputs.shape
    d = token_table.shape[-1]
    n = b * l
    idx_flat = inputs.reshape(n).astype(jnp.int32)
    tok_pad = _transpose_pad(token_table.T)
    pos_pad = jnp.pad(position_table, ((0, 0), (0, _DP - d)))

    grid_kernel = functools.partial(
        pl.kernel,
        mesh=plsc.VectorSubcoreMesh(core_axis_name="c", subcore_axis_name="s"),
        compiler_params=pltpu.CompilerParams(use_tc_tiling_on_sc=True),
        out_type=jax.ShapeDtypeStruct((n, _DP), jnp.float32),
        scratch_types=[
            pltpu.VMEM((n // _NW,), jnp.int32),
            pltpu.VMEM((_NBUF, _CHUNK, _DP), jnp.float32),
            pltpu.VMEM_SHARED((_LREP, _DP), jnp.float32),
            pltpu.SemaphoreType.DMA((_NBUF,)),
            pltpu.SemaphoreType.DMA((_NBUF,)),
            pltpu.SemaphoreType.DMA((_NBUF,)),
        ],
    )(_gather_body)

    out = grid_kernel(idx_flat, tok_pad, pos_pad)
    return out[:, :d].reshape(b, l, d)


# speedup vs baseline: 3.4945x; 1.0007x over previous
import functools

import jax
import jax.numpy as jnp
from jax import lax
from jax.experimental import pallas as pl
from jax.experimental.pallas import tpu as pltpu
from jax.experimental.pallas import tpu_sc as plsc

_L = 200
_D = 64
_DP = 128
_CHUNK = 128
_LREP = 3200
_NBUF = 6
_NC = 2
_NS = 16
_NW = _NC * _NS


def _gather_body(idx_hbm, tok_hbm, pos_hbm, out_hbm,
                 idx_v, obuf_v, pos_sh, psem, gsem, osem):
    n = idx_hbm.shape[0]
    n_per_w = n // _NW
    n_chunks = n_per_w // _CHUNK

    cid = lax.axis_index("c")
    sid = lax.axis_index("s")
    wid = sid * _NC + cid
    base = wid * n_per_w

    pltpu.sync_copy(idx_hbm.at[pl.ds(base, n_per_w)], idx_v)

    for off, ln in ((0, 104), (104, 96)):
        tmp = obuf_v.at[0, pl.ds(0, ln), :]
        pltpu.sync_copy(pos_hbm.at[pl.ds(off, ln), :], tmp)
        pltpu.sync_copy(tmp, pos_sh.at[pl.ds(sid * _L + off, ln), :])

    plsc.subcore_barrier()

    def prefill_start(b, c):
        p0 = lax.rem(c * _CHUNK, _LREP)
        pltpu.make_async_copy(
            pos_sh.at[pl.ds(p0, _CHUNK), :], obuf_v.at[b], psem.at[b]).start()

    def prefill_wait(b):
        pltpu.make_async_copy(
            pos_sh.at[pl.ds(0, _CHUNK), :], obuf_v.at[b], psem.at[b]).wait()

    def gadd_start(b, c):
        pltpu.async_copy(
            tok_hbm.at[idx_v.at[pl.ds(c * _CHUNK, _CHUNK)]],
            obuf_v.at[b], gsem.at[b], add=True)

    def gadd_wait(b):
        pltpu.make_async_copy(
            tok_hbm.at[idx_v.at[pl.ds(0, _CHUNK)]],
            obuf_v.at[b], gsem.at[b]).wait()

    def out_start(b, c):
        pltpu.make_async_copy(
            obuf_v.at[b],
            out_hbm.at[pl.ds(base + c * _CHUNK, _CHUNK), :],
            osem.at[b]).start()

    def out_wait(b):
        pltpu.make_async_copy(
            obuf_v.at[b],
            out_hbm.at[pl.ds(0, _CHUNK), :], osem.at[b]).wait()

    n_steps = n_chunks + 2
    n_groups = (n_steps + _NBUF - 1) // _NBUF

    def group_body(g, carry):
        for b in range(_NBUF):
            i = g * _NBUF + b
            bg = (b + _NBUF - 1) % _NBUF
            bo = (b + _NBUF - 2) % _NBUF

            @pl.when(jnp.logical_and(i >= _NBUF, i < n_chunks))
            def _():
                out_wait(b)

            @pl.when(i < n_chunks)
            def _():
                prefill_start(b, i)

            @pl.when(jnp.logical_and(i >= 1, i <= n_chunks))
            def _():
                prefill_wait(bg)
                gadd_start(bg, i - 1)

            @pl.when(jnp.logical_and(i >= 2, i <= n_chunks + 1))
            def _():
                gadd_wait(bo)
                out_start(bo, i - 2)
        return carry

    lax.fori_loop(0, n_groups, group_body, 0)

    for b in range(_NBUF):
        out_wait(b)


_TBLK = 16384


def _transpose_pad_body(src_ref, dst_ref):
    x = src_ref[...]
    dst_ref[:, 0:_D] = jnp.transpose(x, (1, 0))


def _transpose_pad(tok_t):
    v = tok_t.shape[1]
    grid = pl.cdiv(v, _TBLK)
    return pl.pallas_call(
        _transpose_pad_body,
        grid=(grid,),
        in_specs=[pl.BlockSpec((_D, _TBLK), lambda i: (0, i))],
        out_specs=pl.BlockSpec((_TBLK, _DP), lambda i: (i, 0)),
        out_shape=jax.ShapeDtypeStruct((v, _DP), jnp.float32),
    )(tok_t)


def kernel(inputs, token_table, position_table):
    b, l = inputs.shape
    d = token_table.shape[-1]
    n = b * l
    idx_flat = inputs.reshape(n).astype(jnp.int32)
    tok_pad = _transpose_pad(token_table.T)
    pos_pad = jnp.pad(position_table, ((0, 0), (0, _DP - d)))

    grid_kernel = functools.partial(
        pl.kernel,
        mesh=plsc.VectorSubcoreMesh(core_axis_name="c", subcore_axis_name="s"),
        compiler_params=pltpu.CompilerParams(use_tc_tiling_on_sc=True),
        out_type=jax.ShapeDtypeStruct((n, _DP), jnp.float32),
        scratch_types=[
            pltpu.VMEM((n // _NW,), jnp.int32),
            pltpu.VMEM((_NBUF, _CHUNK, _DP), jnp.float32),
            pltpu.VMEM_SHARED((_LREP, _DP), jnp.float32),
            pltpu.SemaphoreType.DMA((_NBUF,)),
            pltpu.SemaphoreType.DMA((_NBUF,)),
            pltpu.SemaphoreType.DMA((_NBUF,)),
        ],
    )(_gather_body)

    out = grid_kernel(idx_flat, tok_pad, pos_pad)
    return out[:, :d].reshape(b, l, d)


# speedup vs baseline: 3.5594x; 1.0186x over previous
import functools

import jax
import jax.numpy as jnp
from jax import lax
from jax.experimental import pallas as pl
from jax.experimental.pallas import tpu as pltpu
from jax.experimental.pallas import tpu_sc as plsc

_L = 200
_D = 64
_DP = 128
_CHUNK = 128
_LREP = 3200
_NBUF = 6
_NC = 2
_NS = 16
_NW = _NC * _NS


def _gather_body(idx_hbm, tok_hbm, pos_hbm, out_hbm,
                 idx_v, obuf_v, pos_sh, psem, gsem, osem):
    n = idx_hbm.shape[0]
    n_per_w = n // _NW
    n_chunks = n_per_w // _CHUNK

    cid = lax.axis_index("c")
    sid = lax.axis_index("s")
    wid = sid * _NC + cid
    base = wid * n_per_w

    pltpu.sync_copy(idx_hbm.at[pl.ds(base, n_per_w)], idx_v)

    for off, ln in ((0, 104), (104, 96)):
        tmp = obuf_v.at[0, pl.ds(0, ln), :]
        pltpu.sync_copy(pos_hbm.at[pl.ds(off, ln), :], tmp)
        pltpu.sync_copy(tmp, pos_sh.at[pl.ds(sid * _L + off, ln), :])

    plsc.subcore_barrier()

    def prefill_start(b, c):
        p0 = lax.rem(c * _CHUNK, _LREP)
        pltpu.make_async_copy(
            pos_sh.at[pl.ds(p0, _CHUNK), :], obuf_v.at[b], psem.at[b]).start()

    def prefill_wait(b):
        pltpu.make_async_copy(
            pos_sh.at[pl.ds(0, _CHUNK), :], obuf_v.at[b], psem.at[b]).wait()

    def gadd_start(b, c):
        pltpu.async_copy(
            tok_hbm.at[idx_v.at[pl.ds(c * _CHUNK, _CHUNK)]],
            obuf_v.at[b], gsem.at[b], add=True)

    def gadd_wait(b):
        pltpu.make_async_copy(
            tok_hbm.at[idx_v.at[pl.ds(0, _CHUNK)]],
            obuf_v.at[b], gsem.at[b]).wait()

    def out_start(b, c):
        pltpu.make_async_copy(
            obuf_v.at[b],
            out_hbm.at[pl.ds(base + c * _CHUNK, _CHUNK), :],
            osem.at[b]).start()

    def out_wait(b):
        pltpu.make_async_copy(
            obuf_v.at[b],
            out_hbm.at[pl.ds(0, _CHUNK), :], osem.at[b]).wait()

    n_steps = n_chunks + 2
    n_groups = (n_steps + _NBUF - 1) // _NBUF

    def group_body(g, carry):
        for b in range(_NBUF):
            i = g * _NBUF + b
            bg = (b + _NBUF - 1) % _NBUF
            bo = (b + _NBUF - 2) % _NBUF

            @pl.when(jnp.logical_and(i >= _NBUF, i < n_chunks))
            def _():
                out_wait(b)

            @pl.when(i < n_chunks)
            def _():
                prefill_start(b, i)

            @pl.when(jnp.logical_and(i >= 1, i <= n_chunks))
            def _():
                prefill_wait(bg)
                gadd_start(bg, i - 1)

            @pl.when(jnp.logical_and(i >= 2, i <= n_chunks + 1))
            def _():
                gadd_wait(bo)
                out_start(bo, i - 2)
        return carry

    lax.fori_loop(0, n_groups, group_body, 0)

    for b in range(_NBUF):
        out_wait(b)


_TBLK = 32768


def _transpose_pad_body(src_ref, dst_ref):
    x = src_ref[...]
    dst_ref[:, 0:_D] = jnp.transpose(x, (1, 0))


def _transpose_pad(tok_t):
    v = tok_t.shape[1]
    grid = pl.cdiv(v, _TBLK)
    return pl.pallas_call(
        _transpose_pad_body,
        grid=(grid,),
        in_specs=[pl.BlockSpec((_D, _TBLK), lambda i: (0, i))],
        out_specs=pl.BlockSpec((_TBLK, _DP), lambda i: (i, 0)),
        out_shape=jax.ShapeDtypeStruct((v, _DP), jnp.float32),
    )(tok_t)


def kernel(inputs, token_table, position_table):
    b, l = inputs.shape
    d = token_table.shape[-1]
    n = b * l
    idx_flat = inputs.reshape(n).astype(jnp.int32)
    tok_pad = _transpose_pad(token_table.T)
    pos_pad = jnp.pad(position_table, ((0, 0), (0, _DP - d)))

    grid_kernel = functools.partial(
        pl.kernel,
        mesh=plsc.VectorSubcoreMesh(core_axis_name="c", subcore_axis_name="s"),
        compiler_params=pltpu.CompilerParams(use_tc_tiling_on_sc=True),
        out_type=jax.ShapeDtypeStruct((n, _DP), jnp.float32),
        scratch_types=[
            pltpu.VMEM((n // _NW,), jnp.int32),
            pltpu.VMEM((_NBUF, _CHUNK, _DP), jnp.float32),
            pltpu.VMEM_SHARED((_LREP, _DP), jnp.float32),
            pltpu.SemaphoreType.DMA((_NBUF,)),
            pltpu.SemaphoreType.DMA((_NBUF,)),
            pltpu.SemaphoreType.DMA((_NBUF,)),
        ],
    )(_gather_body)

    out = grid_kernel(idx_flat, tok_pad, pos_pad)
    return out[:, :d].reshape(b, l, d)
